# baseline (speedup 1.0000x reference)
import functools

import jax
import jax.numpy as jnp
from jax import lax
from jax.experimental import pallas as pl
from jax.experimental.pallas import tpu as pltpu
from jax.experimental.pallas import tpu_sc as plsc

N_NODES = 10000
N_EDGES = 320000
D = 128

CHUNK = 32
NUM_CHUNKS = N_EDGES // CHUNK
N_CORES = 2
N_SUBCORES = 16
NW = N_CORES * N_SUBCORES
CPW = 320
IDX_ROWS = CPW * CHUNK // 128
CHUNKS_PAD = NW * CPW
E_PAD = CHUNKS_PAD * CHUNK
IDX_ROWS_PAD = E_PAD // 128
N_PAD = 10240
ROWS_PER_TILE = N_PAD // N_SUBCORES
LANES = 16


def _sc_aggregate(x, packed_idx, edge_attr, zeros):
  mesh = plsc.VectorSubcoreMesh(core_axis_name="c", subcore_axis_name="s")

  @functools.partial(
      pl.kernel,
      out_type=jax.ShapeDtypeStruct((N_CORES, N_PAD, D), jnp.float32),
      mesh=mesh,
      scratch_types=[
          pltpu.VMEM((IDX_ROWS, 128), jnp.int32),
          pltpu.VMEM((CHUNK, D), jnp.float32),
          pltpu.VMEM((CHUNK, D), jnp.float32),
          [pltpu.VMEM((CHUNK, D), jnp.float32)] * 4,
          pltpu.VMEM((CHUNK, D), jnp.float32),
          pltpu.VMEM((CHUNK, D), jnp.float32),
          [pltpu.VMEM((CHUNK,), jnp.int32)] * 4,
          [pltpu.VMEM((CHUNK,), jnp.int32)] * 4,
          pltpu.VMEM_SHARED((N_PAD, D), jnp.float32),
          pltpu.SemaphoreType.DMA,
          pltpu.SemaphoreType.DMA,
          [pltpu.SemaphoreType.DMA] * 4,
          pltpu.SemaphoreType.DMA,
          pltpu.SemaphoreType.DMA,
      ],
  )
  def kernel(x_hbm, pk_hbm, ea_hbm, z_hbm, out_hbm,
             win_v, xr0, xr1, er, ms0, ms1, si_ring, di_ring, aggr_sh,
             sgx0, sgx1, sge, ssc0, ssc1):
    c = lax.axis_index("c")
    s = lax.axis_index("s")
    w = c * N_SUBCORES + s
    xr = (xr0, xr1)
    ms = (ms0, ms1)
    sgx = (sgx0, sgx1)
    ssc = (ssc0, ssc1)

    chunk0 = w * CPW
    n_my = jnp.minimum(CPW, NUM_CHUNKS - chunk0)

    pltpu.sync_copy(pk_hbm.at[pl.ds(w * IDX_ROWS, IDX_ROWS)], win_v)

    def stage_idx(row, col, rb):
      for h in range(CHUNK // LANES):
        p = win_v.at[row, pl.ds(col * CHUNK + h * LANES, LANES)][...]
        dsl = pl.ds(h * LANES, LANES)
        si_ring[rb].at[dsl][...] = p & 0xFFFF
        di_ring[rb].at[dsl][...] = lax.shift_right_logical(p, 16)

    def start_gather(rb, db):
      pltpu.async_copy(x_hbm.at[si_ring[rb]], xr[db], sgx[db])

    def wait_gather(rb, db):
      pltpu.make_async_copy(x_hbm.at[si_ring[rb]], xr[db], sgx[db]).wait()

    H = CHUNK // 2

    def start_ea(m, eb):
      base = (chunk0 + m) * CHUNK
      pltpu.async_copy(ea_hbm.at[pl.ds(base, H)],
                       er[eb].at[pl.ds(0, H)], sge[eb])
      pltpu.async_copy(ea_hbm.at[pl.ds(base + H, H)],
                       er[eb].at[pl.ds(H, H)], sge[eb])

    def wait_ea(m, eb):
      base = (chunk0 + m) * CHUNK
      pltpu.make_async_copy(ea_hbm.at[pl.ds(base, H)],
                            er[eb].at[pl.ds(0, H)], sge[eb]).wait()
      pltpu.make_async_copy(ea_hbm.at[pl.ds(base + H, H)],
                            er[eb].at[pl.ds(H, H)], sge[eb]).wait()

    def compute(db, eb):
      @plsc.parallel_loop(0, CHUNK, unroll=4)
      def _(r):
        for jj in range(D // LANES):
          sl = pl.ds(jj * LANES, LANES)
          ms[db].at[r, sl][...] = jnp.maximum(
              xr[db].at[r, sl][...] + er[eb].at[r, sl][...], 0.0)

    def wait_scatter(rb, db):
      pltpu.make_async_copy(ms[db], aggr_sh.at[di_ring[rb]], ssc[db]).wait()

    stage_idx(0, 0, 0)
    stage_idx(0, 1, 1)
    start_gather(0, 0)
    start_gather(1, 1)
    for eb in range(4):
      start_ea(eb, eb)

    pltpu.sync_copy(z_hbm, aggr_sh.at[pl.ds(s * ROWS_PER_TILE, ROWS_PER_TILE)])
    plsc.subcore_barrier()

    @pl.loop(0, n_my, step=4)
    def _(j):
      row = j // 4
      for b in range(4):
        m = j + b
        db = b % 2
        rb2 = (b + 2) % 4

        if b < 2:
          @pl.when(m >= 2)
          def _():
            wait_scatter(rb2, db)
        else:
          wait_scatter(rb2, db)

        wait_gather(b, db)
        wait_ea(m, b)
        compute(db, b)
        pltpu.async_copy(ms[db], aggr_sh.at[di_ring[b]], ssc[db], add=True)

        @pl.when(m + 2 < n_my)
        def _():
          row2 = row + (b + 2) // 4
          stage_idx(row2, rb2, rb2)
          start_gather(rb2, db)

        @pl.when(m + 4 < n_my)
        def _():
          start_ea(m + 4, b)

    wait_scatter(2, 0)
    wait_scatter(3, 1)

    plsc.subcore_barrier()
    row0 = s * ROWS_PER_TILE
    pltpu.sync_copy(aggr_sh.at[pl.ds(row0, ROWS_PER_TILE)],
                    out_hbm.at[c].at[pl.ds(row0, ROWS_PER_TILE)])

  return kernel(x, packed_idx, edge_attr, zeros)


def _tc_mlp(x, aggr, W1, b1, W2, b2):
  BLK = 1000

  def body(x_ref, a_ref, w1_ref, b1_ref, w2_ref, b2_ref, o_ref):
    h = x_ref[...] + a_ref[0] + a_ref[1]
    h = jnp.dot(h, w1_ref[...], preferred_element_type=jnp.float32)
    h = jnp.maximum(h + b1_ref[...], 0.0)
    h = jnp.dot(h, w2_ref[...], preferred_element_type=jnp.float32)
    o_ref[...] = h + b2_ref[...]

  row_spec = pl.BlockSpec((BLK, D), lambda i: (i, 0))
  aggr_spec = pl.BlockSpec((N_CORES, BLK, D), lambda i: (0, i, 0))
  full_spec = pl.BlockSpec((D, D), lambda i: (0, 0))
  bias_spec = pl.BlockSpec((1, D), lambda i: (0, 0))
  return pl.pallas_call(
      body,
      grid=(N_NODES // BLK,),
      in_specs=[row_spec, aggr_spec,
                full_spec, bias_spec, full_spec, bias_spec],
      out_specs=row_spec,
      out_shape=jax.ShapeDtypeStruct((N_NODES, D), jnp.float32),
  )(x, aggr, W1, b1.reshape(1, D), W2, b2.reshape(1, D))


def kernel(x, edge_index, edge_attr, W1, b1, W2, b2):
  src = edge_index[0].astype(jnp.int32)
  dst = edge_index[1].astype(jnp.int32)
  packed = src | (dst << 16)
  pad = E_PAD - N_EDGES
  packed = jnp.pad(packed, (0, pad)).reshape(IDX_ROWS_PAD, 128)
  zeros = jnp.zeros((ROWS_PER_TILE, D), jnp.float32)
  aggr = _sc_aggregate(x, packed, edge_attr, zeros)
  return _tc_mlp(x, aggr, W1, b1, W2, b2)


# speedup vs baseline: 1.1055x; 1.1055x over previous
import dataclasses
import functools

import jax
import jax.numpy as jnp
from jax import lax
from jax.experimental import pallas as pl
from jax.experimental.pallas import tpu as pltpu
from jax.experimental.pallas import tpu_sc as plsc

N_NODES = 10000
N_EDGES = 320000
D = 128

CHUNK = 32
NUM_CHUNKS = N_EDGES // CHUNK
N_CORES = 2
N_SUBCORES = 16
NW = N_CORES * N_SUBCORES
CPW = 320
IDX_ROWS = CPW * CHUNK // 128
CHUNKS_PAD = NW * CPW
E_PAD = CHUNKS_PAD * CHUNK
IDX_ROWS_PAD = E_PAD // 128
N_PAD = 10240
ROWS_PER_TILE = N_PAD // N_SUBCORES
LANES = 16


def _sc_aggregate(x, packed_idx, edge_attr, zeros):
  mesh = plsc.VectorSubcoreMesh(core_axis_name="c", subcore_axis_name="s")
  cp = pltpu.CompilerParams()
  if "needs_layout_passes" in pltpu.CompilerParams.__dataclass_fields__:
    cp = dataclasses.replace(cp, needs_layout_passes=False)
  cp = dataclasses.replace(cp, use_tc_tiling_on_sc=False)

  @functools.partial(
      pl.kernel,
      out_type=jax.ShapeDtypeStruct((N_CORES, N_PAD, D), jnp.float32),
      mesh=mesh,
      compiler_params=cp,
      scratch_types=[
          pltpu.VMEM((IDX_ROWS, 128), jnp.int32),
          pltpu.VMEM((CHUNK, D // 2), jnp.int32),
          pltpu.VMEM((CHUNK, D // 2), jnp.int32),
          [pltpu.VMEM((CHUNK, D), jnp.float32)] * 4,
          pltpu.VMEM((CHUNK, D), jnp.float32),
          pltpu.VMEM((CHUNK, D), jnp.float32),
          [pltpu.VMEM((CHUNK,), jnp.int32)] * 4,
          [pltpu.VMEM((CHUNK,), jnp.int32)] * 4,
          pltpu.VMEM_SHARED((N_PAD, D), jnp.float32),
          pltpu.SemaphoreType.DMA,
          pltpu.SemaphoreType.DMA,
          [pltpu.SemaphoreType.DMA] * 4,
          pltpu.SemaphoreType.DMA,
          pltpu.SemaphoreType.DMA,
      ],
  )
  def kernel(x_hbm, pk_hbm, ea_hbm, z_hbm, out_hbm,
             win_v, xr0, xr1, er, ms0, ms1, si_ring, di_ring, aggr_sh,
             sgx0, sgx1, sge, ssc0, ssc1):
    c = lax.axis_index("c")
    s = lax.axis_index("s")
    w = c * N_SUBCORES + s
    xr = (xr0, xr1)
    ms = (ms0, ms1)
    sgx = (sgx0, sgx1)
    ssc = (ssc0, ssc1)

    chunk0 = w * CPW
    n_my = jnp.minimum(CPW, NUM_CHUNKS - chunk0)

    pltpu.sync_copy(pk_hbm.at[pl.ds(w * IDX_ROWS, IDX_ROWS)], win_v)

    def stage_idx(row, col, rb):
      for h in range(CHUNK // LANES):
        p = win_v.at[row, pl.ds(col * CHUNK + h * LANES, LANES)][...]
        dsl = pl.ds(h * LANES, LANES)
        si_ring[rb].at[dsl][...] = p & 0xFFFF
        di_ring[rb].at[dsl][...] = lax.shift_right_logical(p, 16)

    def start_gather(rb, db):
      pltpu.async_copy(x_hbm.at[si_ring[rb]], xr[db], sgx[db])

    def wait_gather(rb, db):
      pltpu.make_async_copy(x_hbm.at[si_ring[rb]], xr[db], sgx[db]).wait()

    def start_ea(m, eb):
      pltpu.async_copy(ea_hbm.at[pl.ds((chunk0 + m) * CHUNK, CHUNK)],
                       er[eb], sge[eb])

    def wait_ea(m, eb):
      pltpu.make_async_copy(ea_hbm.at[pl.ds((chunk0 + m) * CHUNK, CHUNK)],
                            er[eb], sge[eb]).wait()

    def compute(db, eb):
      @plsc.parallel_loop(0, CHUNK, unroll=4)
      def _(r):
        for jj in range(D // (2 * LANES)):
          xi = xr[db].at[r, pl.ds(jj * LANES, LANES)][...]
          xb = plsc.bitcast(xi, jnp.bfloat16)
          x0, x1 = plsc.unpack(xb, format=plsc.PackFormat.INTERLEAVED)
          sl0 = pl.ds(jj * 2 * LANES, LANES)
          sl1 = pl.ds(jj * 2 * LANES + LANES, LANES)
          ms[db].at[r, sl0][...] = jnp.maximum(
              x0 + er[eb].at[r, sl0][...], 0.0)
          ms[db].at[r, sl1][...] = jnp.maximum(
              x1 + er[eb].at[r, sl1][...], 0.0)

    def wait_scatter(rb, db):
      pltpu.make_async_copy(ms[db], aggr_sh.at[di_ring[rb]], ssc[db]).wait()

    stage_idx(0, 0, 0)
    stage_idx(0, 1, 1)
    start_gather(0, 0)
    start_gather(1, 1)
    for eb in range(4):
      start_ea(eb, eb)

    pltpu.sync_copy(z_hbm, aggr_sh.at[pl.ds(s * ROWS_PER_TILE, ROWS_PER_TILE)])
    plsc.subcore_barrier()

    @pl.loop(0, n_my, step=4)
    def _(j):
      row = j // 4
      for b in range(4):
        m = j + b
        db = b % 2
        rb2 = (b + 2) % 4

        if b < 2:
          @pl.when(m >= 2)
          def _():
            wait_scatter(rb2, db)
        else:
          wait_scatter(rb2, db)

        wait_gather(b, db)
        wait_ea(m, b)
        compute(db, b)
        pltpu.async_copy(ms[db], aggr_sh.at[di_ring[b]], ssc[db], add=True)

        @pl.when(m + 2 < n_my)
        def _():
          row2 = row + (b + 2) // 4
          stage_idx(row2, rb2, rb2)
          start_gather(rb2, db)

        @pl.when(m + 4 < n_my)
        def _():
          start_ea(m + 4, b)

    wait_scatter(2, 0)
    wait_scatter(3, 1)

    plsc.subcore_barrier()
    row0 = s * ROWS_PER_TILE
    pltpu.sync_copy(aggr_sh.at[pl.ds(row0, ROWS_PER_TILE)],
                    out_hbm.at[c].at[pl.ds(row0, ROWS_PER_TILE)])

  return kernel(x, packed_idx, edge_attr, zeros)


def _tc_mlp(x, aggr, W1, b1, W2, b2):
  BLK = 1000

  def body(x_ref, a_ref, w1_ref, b1_ref, w2_ref, b2_ref, o_ref):
    h = x_ref[...] + a_ref[0] + a_ref[1]
    h = jnp.dot(h, w1_ref[...], preferred_element_type=jnp.float32)
    h = jnp.maximum(h + b1_ref[...], 0.0)
    h = jnp.dot(h, w2_ref[...], preferred_element_type=jnp.float32)
    o_ref[...] = h + b2_ref[...]

  row_spec = pl.BlockSpec((BLK, D), lambda i: (i, 0))
  aggr_spec = pl.BlockSpec((N_CORES, BLK, D), lambda i: (0, i, 0))
  full_spec = pl.BlockSpec((D, D), lambda i: (0, 0))
  bias_spec = pl.BlockSpec((1, D), lambda i: (0, 0))
  return pl.pallas_call(
      body,
      grid=(N_NODES // BLK,),
      in_specs=[row_spec, aggr_spec,
                full_spec, bias_spec, full_spec, bias_spec],
      out_specs=row_spec,
      out_shape=jax.ShapeDtypeStruct((N_NODES, D), jnp.float32),
  )(x, aggr, W1, b1.reshape(1, D), W2, b2.reshape(1, D))


def kernel(x, edge_index, edge_attr, W1, b1, W2, b2):
  src = edge_index[0].astype(jnp.int32)
  dst = edge_index[1].astype(jnp.int32)
  packed = src | (dst << 16)
  pad = E_PAD - N_EDGES
  packed = jnp.pad(packed, (0, pad)).reshape(IDX_ROWS_PAD, 128)
  zeros = jnp.zeros((ROWS_PER_TILE, D), jnp.float32)
  x_bf = (x.astype(jnp.bfloat16)
          .reshape(N_NODES, D // 32, 2, 16)
          .transpose(0, 1, 3, 2)
          .reshape(N_NODES, D // 2, 2))
  x_pk = jax.lax.bitcast_convert_type(x_bf, jnp.int32)
  aggr = _sc_aggregate(x_pk, packed, edge_attr, zeros)
  return _tc_mlp(x, aggr, W1, b1, W2, b2)
